# baseline (speedup 1.0000x reference)
import functools

import jax
import jax.numpy as jnp
from jax.experimental import pallas as pl
from jax.experimental.pallas import tpu as pltpu

_TB = 256
_RPB = 32


def _dan_kernel(ids_smem, emb_hbm, w1_ref, b1_ref, w2_ref, b2_ref,
                out_ref, e3_ref, mean_ref, sem, *, seq_len, tb):
    gi = pl.program_id(0)

    @pl.when(gi == 0)
    def _load_table():
        v = e3_ref.shape[0]
        vq = v // 4
        starts = [0, vq, 2 * vq, 3 * vq]
        sizes = [vq, vq, vq, v - 3 * vq]
        cps = [pltpu.make_async_copy(
                   emb_hbm.at[pl.ds(starts[k], sizes[k]), :],
                   e3_ref.at[pl.ds(starts[k], sizes[k]), 0, :],
                   sem.at[k])
               for k in range(4)]
        for cp in cps:
            cp.start()
        for cp in cps:
            cp.wait()

    nw = (seq_len + 1) // 2
    odd = seq_len % 2

    def body(g, carry):
        row0 = g * _RPB
        base = (gi * tb + row0) * nw
        for r in range(_RPB):
            rowbase = base + r * nw
            w0 = ids_smem[rowbase]
            acc = e3_ref[pl.ds(w0 & 0xFFFF, 1), 0, :]
            if seq_len > 1:
                acc = acc + e3_ref[
                    pl.ds(jax.lax.shift_right_logical(w0, 16), 1), 0, :]
            for sp in range(1, nw):
                w = ids_smem[rowbase + sp]
                acc = acc + e3_ref[pl.ds(w & 0xFFFF, 1), 0, :]
                if sp < nw - 1 or not odd:
                    acc = acc + e3_ref[
                        pl.ds(jax.lax.shift_right_logical(w, 16), 1), 0, :]
            mean_ref[row0 + r, 0, :] = acc[0, :]
        return carry

    jax.lax.fori_loop(0, tb // _RPB, body, 0)

    mean = mean_ref[...].reshape(tb, mean_ref.shape[2]) * (1.0 / seq_len)
    h = jnp.dot(mean, w1_ref[...],
                preferred_element_type=jnp.float32) + b1_ref[...]
    h = jnp.maximum(h, 0.0)
    logits = jnp.dot(h, w2_ref[...],
                     preferred_element_type=jnp.float32) + b2_ref[...]
    m = jnp.max(logits, axis=1, keepdims=True)
    shifted = logits - m
    lse = jnp.log(jnp.sum(jnp.exp(shifted), axis=1, keepdims=True))
    out_ref[...] = shifted - lse


def kernel(ids, emb, w1, b1, w2, b2):
    B, S = ids.shape
    V, D = emb.shape
    H = w1.shape[1]
    O = w2.shape[1]

    nb = pl.cdiv(B, _TB)
    Bp = nb * _TB
    ids_p = ids
    if Bp != B:
        ids_p = jnp.zeros((Bp, S), jnp.int32).at[:B, :].set(ids)
    if S % 2:
        ids_p = jnp.concatenate(
            [ids_p, jnp.zeros((Bp, 1), jnp.int32)], axis=1)
    ids_pack = (ids_p[:, 0::2] | (ids_p[:, 1::2] << 16)).reshape(-1)

    out = pl.pallas_call(
        functools.partial(_dan_kernel, seq_len=S, tb=_TB),
        out_shape=jax.ShapeDtypeStruct((Bp, O), jnp.float32),
        grid=(nb,),
        in_specs=[
            pl.BlockSpec(memory_space=pltpu.SMEM),
            pl.BlockSpec(memory_space=pl.ANY),
            pl.BlockSpec((D, H), lambda i: (0, 0)),
            pl.BlockSpec((1, H), lambda i: (0, 0)),
            pl.BlockSpec((H, O), lambda i: (0, 0)),
            pl.BlockSpec((1, O), lambda i: (0, 0)),
        ],
        out_specs=pl.BlockSpec((_TB, O), lambda i: (i, 0)),
        scratch_shapes=[pltpu.VMEM((V, 1, D), jnp.float32),
                        pltpu.VMEM((_TB, 1, D), jnp.float32),
                        pltpu.SemaphoreType.DMA((4,))],
        compiler_params=pltpu.CompilerParams(
            dimension_semantics=("arbitrary",)),
    )(ids_pack, emb, w1, b1, w2, b2)

    return out[:B, :]


# speedup vs baseline: 1.0130x; 1.0130x over previous
import functools

import jax
import jax.numpy as jnp
from jax.experimental import pallas as pl
from jax.experimental.pallas import tpu as pltpu

_TB = 512
_RPB = 32


def _dan_kernel(ids_smem, emb_hbm, w1_ref, b1_ref, w2_ref, b2_ref,
                out_ref, e3_ref, mean_ref, sem, *, seq_len, tb):
    gi = pl.program_id(0)

    @pl.when(gi == 0)
    def _load_table():
        v = e3_ref.shape[0]
        vq = v // 4
        starts = [0, vq, 2 * vq, 3 * vq]
        sizes = [vq, vq, vq, v - 3 * vq]
        cps = [pltpu.make_async_copy(
                   emb_hbm.at[pl.ds(starts[k], sizes[k]), :],
                   e3_ref.at[pl.ds(starts[k], sizes[k]), 0, :],
                   sem.at[k])
               for k in range(4)]
        for cp in cps:
            cp.start()
        for cp in cps:
            cp.wait()

    nw = (seq_len + 1) // 2
    odd = seq_len % 2

    def body(g, carry):
        row0 = g * _RPB
        base = (gi * tb + row0) * nw
        for r in range(_RPB):
            rowbase = base + r * nw
            w0 = ids_smem[rowbase]
            acc = e3_ref[pl.ds(w0 & 0xFFFF, 1), 0, :]
            if seq_len > 1:
                acc = acc + e3_ref[
                    pl.ds(jax.lax.shift_right_logical(w0, 16), 1), 0, :]
            for sp in range(1, nw):
                w = ids_smem[rowbase + sp]
                acc = acc + e3_ref[pl.ds(w & 0xFFFF, 1), 0, :]
                if sp < nw - 1 or not odd:
                    acc = acc + e3_ref[
                        pl.ds(jax.lax.shift_right_logical(w, 16), 1), 0, :]
            mean_ref[row0 + r, 0, :] = acc[0, :]
        return carry

    jax.lax.fori_loop(0, tb // _RPB, body, 0)

    mean = mean_ref[...].reshape(tb, mean_ref.shape[2]) * (1.0 / seq_len)
    h = jnp.dot(mean, w1_ref[...],
                preferred_element_type=jnp.float32) + b1_ref[...]
    h = jnp.maximum(h, 0.0)
    logits = jnp.dot(h, w2_ref[...],
                     preferred_element_type=jnp.float32) + b2_ref[...]
    m = jnp.max(logits, axis=1, keepdims=True)
    shifted = logits - m
    lse = jnp.log(jnp.sum(jnp.exp(shifted), axis=1, keepdims=True))
    out_ref[...] = shifted - lse


def kernel(ids, emb, w1, b1, w2, b2):
    B, S = ids.shape
    V, D = emb.shape
    H = w1.shape[1]
    O = w2.shape[1]

    nb = pl.cdiv(B, _TB)
    Bp = nb * _TB
    ids_p = ids
    if Bp != B:
        ids_p = jnp.zeros((Bp, S), jnp.int32).at[:B, :].set(ids)
    if S % 2:
        ids_p = jnp.concatenate(
            [ids_p, jnp.zeros((Bp, 1), jnp.int32)], axis=1)
    ids_pack = (ids_p[:, 0::2] | (ids_p[:, 1::2] << 16)).reshape(-1)

    out = pl.pallas_call(
        functools.partial(_dan_kernel, seq_len=S, tb=_TB),
        out_shape=jax.ShapeDtypeStruct((Bp, O), jnp.float32),
        grid=(nb,),
        in_specs=[
            pl.BlockSpec(memory_space=pltpu.SMEM),
            pl.BlockSpec(memory_space=pl.ANY),
            pl.BlockSpec((D, H), lambda i: (0, 0)),
            pl.BlockSpec((1, H), lambda i: (0, 0)),
            pl.BlockSpec((H, O), lambda i: (0, 0)),
            pl.BlockSpec((1, O), lambda i: (0, 0)),
        ],
        out_specs=pl.BlockSpec((_TB, O), lambda i: (i, 0)),
        scratch_shapes=[pltpu.VMEM((V, 1, D), jnp.float32),
                        pltpu.VMEM((_TB, 1, D), jnp.float32),
                        pltpu.SemaphoreType.DMA((4,))],
        compiler_params=pltpu.CompilerParams(
            dimension_semantics=("arbitrary",)),
    )(ids_pack, emb, w1, b1, w2, b2)

    return out[:B, :]


# speedup vs baseline: 1.0165x; 1.0034x over previous
import functools

import jax
import jax.numpy as jnp
from jax.experimental import pallas as pl
from jax.experimental.pallas import tpu as pltpu

_TB = 1024
_RPB = 32


def _dan_kernel(ids_smem, emb_hbm, w1_ref, b1_ref, w2_ref, b2_ref,
                out_ref, e3_ref, mean_ref, sem, *, seq_len, tb):
    gi = pl.program_id(0)

    @pl.when(gi == 0)
    def _load_table():
        v = e3_ref.shape[0]
        vq = v // 4
        starts = [0, vq, 2 * vq, 3 * vq]
        sizes = [vq, vq, vq, v - 3 * vq]
        cps = [pltpu.make_async_copy(
                   emb_hbm.at[pl.ds(starts[k], sizes[k]), :],
                   e3_ref.at[pl.ds(starts[k], sizes[k]), 0, :],
                   sem.at[k])
               for k in range(4)]
        for cp in cps:
            cp.start()
        for cp in cps:
            cp.wait()

    nw = (seq_len + 1) // 2
    odd = seq_len % 2

    def body(g, carry):
        row0 = g * _RPB
        base = (gi * tb + row0) * nw
        for r in range(_RPB):
            rowbase = base + r * nw
            w0 = ids_smem[rowbase]
            acc = e3_ref[pl.ds(w0 & 0xFFFF, 1), 0, :]
            if seq_len > 1:
                acc = acc + e3_ref[
                    pl.ds(jax.lax.shift_right_logical(w0, 16), 1), 0, :]
            for sp in range(1, nw):
                w = ids_smem[rowbase + sp]
                acc = acc + e3_ref[pl.ds(w & 0xFFFF, 1), 0, :]
                if sp < nw - 1 or not odd:
                    acc = acc + e3_ref[
                        pl.ds(jax.lax.shift_right_logical(w, 16), 1), 0, :]
            mean_ref[row0 + r, 0, :] = acc[0, :]
        return carry

    jax.lax.fori_loop(0, tb // _RPB, body, 0)

    mean = mean_ref[...].reshape(tb, mean_ref.shape[2]) * (1.0 / seq_len)
    h = jnp.dot(mean, w1_ref[...],
                preferred_element_type=jnp.float32) + b1_ref[...]
    h = jnp.maximum(h, 0.0)
    logits = jnp.dot(h, w2_ref[...],
                     preferred_element_type=jnp.float32) + b2_ref[...]
    m = jnp.max(logits, axis=1, keepdims=True)
    shifted = logits - m
    lse = jnp.log(jnp.sum(jnp.exp(shifted), axis=1, keepdims=True))
    out_ref[...] = shifted - lse


def kernel(ids, emb, w1, b1, w2, b2):
    B, S = ids.shape
    V, D = emb.shape
    H = w1.shape[1]
    O = w2.shape[1]

    nb = pl.cdiv(B, _TB)
    Bp = nb * _TB
    ids_p = ids
    if Bp != B:
        ids_p = jnp.zeros((Bp, S), jnp.int32).at[:B, :].set(ids)
    if S % 2:
        ids_p = jnp.concatenate(
            [ids_p, jnp.zeros((Bp, 1), jnp.int32)], axis=1)
    ids_pack = (ids_p[:, 0::2] | (ids_p[:, 1::2] << 16)).reshape(-1)

    out = pl.pallas_call(
        functools.partial(_dan_kernel, seq_len=S, tb=_TB),
        out_shape=jax.ShapeDtypeStruct((Bp, O), jnp.float32),
        grid=(nb,),
        in_specs=[
            pl.BlockSpec(memory_space=pltpu.SMEM),
            pl.BlockSpec(memory_space=pl.ANY),
            pl.BlockSpec((D, H), lambda i: (0, 0)),
            pl.BlockSpec((1, H), lambda i: (0, 0)),
            pl.BlockSpec((H, O), lambda i: (0, 0)),
            pl.BlockSpec((1, O), lambda i: (0, 0)),
        ],
        out_specs=pl.BlockSpec((_TB, O), lambda i: (i, 0)),
        scratch_shapes=[pltpu.VMEM((V, 1, D), jnp.float32),
                        pltpu.VMEM((_TB, 1, D), jnp.float32),
                        pltpu.SemaphoreType.DMA((4,))],
        compiler_params=pltpu.CompilerParams(
            dimension_semantics=("arbitrary",)),
    )(ids_pack, emb, w1, b1, w2, b2)

    return out[:B, :]
